# baseline (speedup 1.0000x reference)
import functools

import jax
import jax.numpy as jnp
from jax import lax
from jax.experimental import pallas as pl
from jax.experimental.pallas import tpu as pltpu
from jax.experimental.pallas import tpu_sc as plsc


@functools.partial(jax.jit, static_argnames=("n",))
def _broadcast_sc(pos_embed, n):
    seq_len, d_model = pos_embed.shape
    info = plsc.get_sparse_core_info()
    num_workers = info.num_cores * info.num_subcores
    assert seq_len % num_workers == 0
    rows = seq_len // num_workers

    emb3 = pos_embed.reshape(seq_len, 1, d_model)
    mesh = plsc.VectorSubcoreMesh(core_axis_name="c", subcore_axis_name="s")

    @functools.partial(
        pl.kernel,
        mesh=mesh,
        out_type=jax.ShapeDtypeStruct((seq_len, n, d_model), jnp.float32),
        scratch_types=[pltpu.SemaphoreType.DMA],
    )
    def k(emb_hbm, out_hbm, sem):
        wid = lax.axis_index("s") * info.num_cores + lax.axis_index("c")
        base = wid * rows
        copies = [
            pltpu.async_copy(
                emb_hbm.at[pl.ds(base, rows)],
                out_hbm.at[pl.ds(base, rows), pl.ds(j, 1)],
                sem,
            )
            for j in range(n)
        ]
        for c in copies:
            c.wait()

    return k(emb3)


def kernel(z, pos_embed):
    if z.ndim == 2:
        n = z.shape[0]
    elif z.ndim == 3:
        n = z.shape[1]
    else:
        raise Exception
    return _broadcast_sc(pos_embed, n)


# speedup vs baseline: 24.0082x; 24.0082x over previous
import functools

import jax
import jax.numpy as jnp
from jax import lax
from jax.experimental import pallas as pl
from jax.experimental.pallas import tpu as pltpu
from jax.experimental.pallas import tpu_sc as plsc


@functools.partial(jax.jit, static_argnames=("n",))
def _broadcast_sc(pos_embed, n):
    seq_len, d_model = pos_embed.shape
    info = plsc.get_sparse_core_info()
    nc, ns = info.num_cores, info.num_subcores
    assert seq_len % (nc * ns) == 0
    rows_sc = seq_len // nc
    rows = rows_sc // ns

    emb3 = pos_embed.reshape(seq_len, 1, d_model)
    mesh = plsc.VectorSubcoreMesh(core_axis_name="c", subcore_axis_name="s")

    @functools.partial(
        pl.kernel,
        mesh=mesh,
        out_type=jax.ShapeDtypeStruct((seq_len, n, d_model), jnp.float32),
        scratch_types=[
            pltpu.VMEM_SHARED((rows_sc, 1, d_model), jnp.float32),
            pltpu.SemaphoreType.DMA,
        ],
    )
    def k(emb_hbm, out_hbm, shared, wsem):
        cid = lax.axis_index("c")
        sid = lax.axis_index("s")
        base = cid * rows_sc + sid * rows
        lbase = sid * rows
        pltpu.sync_copy(emb_hbm.at[pl.ds(base, rows)], shared.at[pl.ds(lbase, rows)])
        copies = [
            pltpu.async_copy(
                shared.at[pl.ds(lbase, rows)],
                out_hbm.at[pl.ds(base, rows), pl.ds(j, 1)],
                wsem,
            )
            for j in range(n)
        ]
        for c in copies:
            c.wait()

    return k(emb3)


def kernel(z, pos_embed):
    if z.ndim == 2:
        n = z.shape[0]
    elif z.ndim == 3:
        n = z.shape[1]
    else:
        raise Exception
    return _broadcast_sc(pos_embed, n)


# speedup vs baseline: 27.0154x; 1.1253x over previous
import functools

import jax
import jax.numpy as jnp
from jax import lax
from jax.experimental import pallas as pl
from jax.experimental.pallas import tpu as pltpu
from jax.experimental.pallas import tpu_sc as plsc


@functools.partial(jax.jit, static_argnames=("n",))
def _broadcast_sc(pos_embed, n):
    seq_len, d_model = pos_embed.shape
    info = plsc.get_sparse_core_info()
    nc, ns = info.num_cores, info.num_subcores
    assert seq_len % (nc * ns * 2) == 0
    rows_sc = seq_len // nc
    rows = rows_sc // ns
    half = rows // 2

    emb3 = pos_embed.reshape(seq_len, 1, d_model)
    mesh = plsc.VectorSubcoreMesh(core_axis_name="c", subcore_axis_name="s")

    @functools.partial(
        pl.kernel,
        mesh=mesh,
        out_type=jax.ShapeDtypeStruct((seq_len, n, d_model), jnp.float32),
        scratch_types=[
            pltpu.VMEM((half, 1, d_model), jnp.float32),
            pltpu.VMEM_SHARED((ns * half, 1, d_model), jnp.float32),
            pltpu.SemaphoreType.DMA,
            pltpu.SemaphoreType.DMA,
            pltpu.SemaphoreType.DMA,
        ],
    )
    def k(emb_hbm, out_hbm, buf, shared, rsem1, rsem2, wsem):
        cid = lax.axis_index("c")
        sid = lax.axis_index("s")
        base = cid * rows_sc + sid * rows
        lbase = sid * half
        r1 = pltpu.async_copy(emb_hbm.at[pl.ds(base, half)], buf, rsem1)
        r2 = pltpu.async_copy(
            emb_hbm.at[pl.ds(base + half, half)],
            shared.at[pl.ds(lbase, half)],
            rsem2,
        )
        writes = []
        r1.wait()
        for j in range(n):
            writes.append(
                pltpu.async_copy(
                    buf, out_hbm.at[pl.ds(base, half), pl.ds(j, 1)], wsem
                )
            )
        r2.wait()
        for j in range(n):
            writes.append(
                pltpu.async_copy(
                    shared.at[pl.ds(lbase, half)],
                    out_hbm.at[pl.ds(base + half, half), pl.ds(j, 1)],
                    wsem,
                )
            )
        for w in writes:
            w.wait()

    return k(emb3)


def kernel(z, pos_embed):
    if z.ndim == 2:
        n = z.shape[0]
    elif z.ndim == 3:
        n = z.shape[1]
    else:
        raise Exception
    return _broadcast_sc(pos_embed, n)


# speedup vs baseline: 31.6126x; 1.1702x over previous
import functools

import jax
import jax.numpy as jnp
from jax import lax
from jax.experimental import pallas as pl
from jax.experimental.pallas import tpu as pltpu
from jax.experimental.pallas import tpu_sc as plsc


@functools.partial(jax.jit, static_argnames=("n",))
def _broadcast_sc(pos_embed, n):
    seq_len, d_model = pos_embed.shape
    info = plsc.get_sparse_core_info()
    num_workers = info.num_cores * info.num_subcores
    assert seq_len % num_workers == 0
    rows = seq_len // num_workers

    mesh = plsc.VectorSubcoreMesh(core_axis_name="c", subcore_axis_name="s")

    @functools.partial(
        pl.kernel,
        mesh=mesh,
        out_type=jax.ShapeDtypeStruct((seq_len, n, d_model), jnp.float32),
        scratch_types=[
            pltpu.VMEM((rows, d_model), jnp.float32),
            pltpu.SemaphoreType.DMA,
        ],
    )
    def k(emb_hbm, out_hbm, buf, sem):
        wid = lax.axis_index("s") * info.num_cores + lax.axis_index("c")
        base = wid * rows
        pltpu.sync_copy(emb_hbm.at[pl.ds(base, rows)], buf)
        copies = [
            pltpu.async_copy(buf, out_hbm.at[pl.ds(base, rows), j], sem)
            for j in range(n)
        ]
        for c in copies:
            c.wait()

    return k(pos_embed)


def kernel(z, pos_embed):
    if z.ndim == 2:
        n = z.shape[0]
    elif z.ndim == 3:
        n = z.shape[1]
    else:
        raise Exception
    return _broadcast_sc(pos_embed, n)
